# Initial kernel scaffold; baseline (speedup 1.0000x reference)
import jax
import jax.numpy as jnp
from jax.experimental import pallas as pl


def kernel(unit_nums, embed_weight):
    raise NotImplementedError("write your pallas kernel here")


# speedup vs baseline: 194.5988x; 194.5988x over previous
import functools

import jax
import jax.numpy as jnp
from jax import lax
from jax.experimental import pallas as pl
from jax.experimental.pallas import tpu as pltpu
from jax.experimental.pallas import tpu_sc as plsc

_N_UNIT = 100
_MAX_UNIT_SIZE = 800
_TABLE = _N_UNIT * _MAX_UNIT_SIZE
_B = 16384
_TOTAL = _B * _N_UNIT
_NC = 2
_NS = 16
_NW = _NC * _NS
_PER_W = _TOTAL // _NW
_L = 16
_PERIOD = 400
_CHUNK = 6400
_NCHUNK = _PER_W // _CHUNK


_BATCH = 5


def _body(nums_hbm, table_hbm, out_hbm, table_v, x0, x1, o0, o1, shared,
          sin0, sin1, sout0, sout1):
    cid = lax.axis_index("c")
    sid = lax.axis_index("s")
    wid = sid * _NC + cid
    base = wid * _PER_W

    xbufs = (x0, x1)
    obufs = (o0, o1)
    sins = (sin0, sin1)
    souts = (sout0, sout1)

    in_h = [None] * _NCHUNK
    out_h = [None] * _NCHUNK
    for c in range(min(2, _NCHUNK)):
        in_h[c] = pltpu.async_copy(
            nums_hbm.at[pl.ds(base + c * _CHUNK, _CHUNK)], xbufs[c], sins[c]
        )

    @pl.when(sid == 0)
    def _():
        pltpu.sync_copy(table_hbm, shared)

    plsc.subcore_barrier()
    pltpu.sync_copy(shared, table_v)

    iota = lax.broadcasted_iota(jnp.int32, (_L,), 0)

    zeros = jnp.zeros((_L,), jnp.float32)
    for g in range(7):
        unit = iota + g * _L
        plsc.store_scatter(
            table_v, [unit * _MAX_UNIT_SIZE], zeros, mask=unit < _N_UNIT
        )

    shifts = []
    for j in range(_PERIOD // _L):
        col = (iota + j * _L) % _N_UNIT
        shifts.append(col * _MAX_UNIT_SIZE)

    def compute(xb, ob):
        @plsc.parallel_loop(0, _CHUNK // _PERIOD, 1, unroll=1)
        def _(g):
            b0 = g * _PERIOD
            for blk in range(_PERIOD // _L // _BATCH):
                js = range(blk * _BATCH, (blk + 1) * _BATCH)
                offs = [pl.ds(b0 + j * _L, _L) for j in js]
                xs = [xb[o] for o in offs]
                idxs = [x + shifts[j] for x, j in zip(xs, js)]
                vs = [plsc.load_gather(table_v, [i]) for i in idxs]
                for o, v in zip(offs, vs):
                    ob[o] = jnp.maximum(v, 0.0)

    for c in range(_NCHUNK):
        s = c % 2
        in_h[c].wait()
        if c >= 2:
            out_h[c - 2].wait()
        compute(xbufs[s], obufs[s])
        out_h[c] = pltpu.async_copy(
            obufs[s], out_hbm.at[pl.ds(base + c * _CHUNK, _CHUNK)], souts[s]
        )
        if c + 2 < _NCHUNK:
            in_h[c + 2] = pltpu.async_copy(
                nums_hbm.at[pl.ds(base + (c + 2) * _CHUNK, _CHUNK)],
                xbufs[s],
                sins[s],
            )
    out_h[_NCHUNK - 2].wait()
    out_h[_NCHUNK - 1].wait()


@jax.jit
def _run(nums_flat, table_flat):
    mesh = plsc.VectorSubcoreMesh(core_axis_name="c", subcore_axis_name="s")
    return pl.kernel(
        _body,
        mesh=mesh,
        compiler_params=pltpu.CompilerParams(needs_layout_passes=False),
        out_type=jax.ShapeDtypeStruct((_TOTAL,), jnp.float32),
        scratch_types=[
            pltpu.VMEM((_TABLE,), jnp.float32),
            pltpu.VMEM((_CHUNK,), jnp.int32),
            pltpu.VMEM((_CHUNK,), jnp.int32),
            pltpu.VMEM((_CHUNK,), jnp.float32),
            pltpu.VMEM((_CHUNK,), jnp.float32),
            pltpu.VMEM_SHARED((_TABLE,), jnp.float32),
            pltpu.SemaphoreType.DMA,
            pltpu.SemaphoreType.DMA,
            pltpu.SemaphoreType.DMA,
            pltpu.SemaphoreType.DMA,
        ],
    )(nums_flat, table_flat)


def kernel(unit_nums, embed_weight):
    out = _run(unit_nums.reshape(-1), embed_weight.reshape(-1))
    return out.reshape(_B, _N_UNIT)


# speedup vs baseline: 323.4705x; 1.6622x over previous
import functools

import jax
import jax.numpy as jnp
from jax import lax
from jax.experimental import pallas as pl
from jax.experimental.pallas import tpu as pltpu
from jax.experimental.pallas import tpu_sc as plsc

_N_UNIT = 100
_MAX_UNIT_SIZE = 800
_TABLE = _N_UNIT * _MAX_UNIT_SIZE
_B = 16384
_NC = 2
_NS = 16
_NW = _NC * _NS
_ROWS_W = _B // _NW
_L = 16
_CHUNK_R = 64
_NCHUNK = _ROWS_W // _CHUNK_R
_COLS = (0, 16, 32, 48, 64, 80, 84)


def _body(nums_hbm, table_hbm, out_hbm, table_v, x0, x1, o0, o1, shared,
          sin0, sin1, sout0, sout1):
    cid = lax.axis_index("c")
    sid = lax.axis_index("s")
    wid = sid * _NC + cid
    base = wid * _ROWS_W

    xbufs = (x0, x1)
    obufs = (o0, o1)
    sins = (sin0, sin1)
    souts = (sout0, sout1)

    in_h = [None] * _NCHUNK
    out_h = [None] * _NCHUNK
    for c in range(min(2, _NCHUNK)):
        in_h[c] = pltpu.async_copy(
            nums_hbm.at[pl.ds(base + c * _CHUNK_R, _CHUNK_R), :],
            xbufs[c],
            sins[c],
        )

    @pl.when(sid == 0)
    def _():
        pltpu.sync_copy(table_hbm, shared)

    plsc.subcore_barrier()
    pltpu.sync_copy(shared, table_v)

    iota = lax.broadcasted_iota(jnp.int32, (_L,), 0)

    zeros = jnp.zeros((_L,), jnp.float32)
    for g in range(7):
        unit = iota + g * _L
        plsc.store_scatter(
            table_v, [unit * _MAX_UNIT_SIZE], zeros, mask=unit < _N_UNIT
        )

    shifts = [(iota + c) * _MAX_UNIT_SIZE for c in _COLS]

    def compute(xb, ob):
        @plsc.parallel_loop(0, _CHUNK_R, 1, unroll=2)
        def _(r):
            offs = [pl.ds(c, _L) for c in _COLS]
            xs = [xb[r, o] for o in offs]
            idxs = [x + s for x, s in zip(xs, shifts)]
            vs = [plsc.load_gather(table_v, [i]) for i in idxs]
            for o, v in zip(offs, vs):
                ob[r, o] = jnp.maximum(v, 0.0)

    for c in range(_NCHUNK):
        s = c % 2
        in_h[c].wait()
        if c >= 2:
            out_h[c - 2].wait()
        compute(xbufs[s], obufs[s])
        out_h[c] = pltpu.async_copy(
            obufs[s],
            out_hbm.at[pl.ds(base + c * _CHUNK_R, _CHUNK_R), :],
            souts[s],
        )
        if c + 2 < _NCHUNK:
            in_h[c + 2] = pltpu.async_copy(
                nums_hbm.at[pl.ds(base + (c + 2) * _CHUNK_R, _CHUNK_R), :],
                xbufs[s],
                sins[s],
            )
    out_h[_NCHUNK - 2].wait()
    out_h[_NCHUNK - 1].wait()


@jax.jit
def _run(unit_nums, table_flat):
    mesh = plsc.VectorSubcoreMesh(core_axis_name="c", subcore_axis_name="s")
    return pl.kernel(
        _body,
        mesh=mesh,
        compiler_params=pltpu.CompilerParams(needs_layout_passes=False),
        out_type=jax.ShapeDtypeStruct((_B, _N_UNIT), jnp.float32),
        scratch_types=[
            pltpu.VMEM((_TABLE,), jnp.float32),
            pltpu.VMEM((_CHUNK_R, _N_UNIT), jnp.int32),
            pltpu.VMEM((_CHUNK_R, _N_UNIT), jnp.int32),
            pltpu.VMEM((_CHUNK_R, _N_UNIT), jnp.float32),
            pltpu.VMEM((_CHUNK_R, _N_UNIT), jnp.float32),
            pltpu.VMEM_SHARED((_TABLE,), jnp.float32),
            pltpu.SemaphoreType.DMA,
            pltpu.SemaphoreType.DMA,
            pltpu.SemaphoreType.DMA,
            pltpu.SemaphoreType.DMA,
        ],
    )(unit_nums, table_flat)


def kernel(unit_nums, embed_weight):
    return _run(unit_nums, embed_weight.reshape(-1))


# speedup vs baseline: 468.9269x; 1.4497x over previous
import functools

import jax
import jax.numpy as jnp
from jax import lax
from jax.experimental import pallas as pl
from jax.experimental.pallas import tpu as pltpu
from jax.experimental.pallas import tpu_sc as plsc

_N_UNIT = 100
_MAX_UNIT_SIZE = 800
_TABLE = _N_UNIT * _MAX_UNIT_SIZE
_B = 16384
_NC = 2
_NS = 16
_NW = _NC * _NS
_COLS_W = _B // _NW
_L = 16
_CH = 128
_NCH = _COLS_W // _CH
_R1 = 48
_R2 = _N_UNIT - _R1
_SPLIT = _R1 * _MAX_UNIT_SIZE


def _body(nums_hbm, table_hbm, out_hbm, tv1, tv2, x0, x1, o0, o1, shared,
          sin0, sin1, sout0, sout1, semt):
    cid = lax.axis_index("c")
    sid = lax.axis_index("s")
    wid = sid * _NC + cid
    base = wid * _COLS_W

    xbufs = (x0, x1)
    obufs = (o0, o1)
    sins = (sin0, sin1)
    souts = (sout0, sout1)

    rows = (_R1, _R2)

    def in_copy(c, buf, sem):
        phase, cc = divmod(c, _NCH)
        r0 = phase * _R1
        nr = rows[phase]
        return pltpu.async_copy(
            nums_hbm.at[pl.ds(r0, nr), pl.ds(base + cc * _CH, _CH)],
            buf.at[pl.ds(0, nr), :],
            sem,
        )

    nchunks = 2 * _NCH
    in_h = [None] * nchunks
    out_h = [None] * nchunks
    for c in range(2):
        in_h[c] = in_copy(c, xbufs[c], sins[c])

    @pl.when(sid == 0)
    def _():
        pltpu.sync_copy(table_hbm, shared)

    plsc.subcore_barrier()
    pltpu.sync_copy(shared.at[pl.ds(0, _SPLIT)], tv1)
    h2 = pltpu.async_copy(shared.at[pl.ds(_SPLIT, _TABLE - _SPLIT)], tv2, semt)

    iota = lax.broadcasted_iota(jnp.int32, (_L,), 0)
    zeros = jnp.zeros((_L,), jnp.float32)

    def zero_mask_slots(tv, nr):
        for g in range((nr + _L - 1) // _L):
            unit = iota + g * _L
            plsc.store_scatter(
                tv, [unit * _MAX_UNIT_SIZE], zeros, mask=unit < nr
            )

    zero_mask_slots(tv1, _R1)

    def compute(xb, ob, tv, nr):
        @plsc.parallel_loop(0, nr, 1, unroll=2)
        def _(r):
            shift = r * _MAX_UNIT_SIZE
            for k in range(_CH // _L):
                off = pl.ds(k * _L, _L)
                idx = xb[r, off] + shift
                v = plsc.load_gather(tv, [idx])
                ob[r, off] = jnp.maximum(v, 0.0)

    for c in range(nchunks):
        s = c % 2
        phase, cc = divmod(c, _NCH)
        if c == _NCH:
            h2.wait()
            zero_mask_slots(tv2, _R2)
        nr = rows[phase]
        r0 = phase * _R1
        in_h[c].wait()
        if c >= 2:
            out_h[c - 2].wait()
        compute(xbufs[s], obufs[s], tv1 if phase == 0 else tv2, nr)
        out_h[c] = pltpu.async_copy(
            obufs[s].at[pl.ds(0, nr), :],
            out_hbm.at[pl.ds(r0, nr), pl.ds(base + cc * _CH, _CH)],
            souts[s],
        )
        if c + 2 < nchunks:
            in_h[c + 2] = in_copy(c + 2, xbufs[s], sins[s])
    out_h[nchunks - 2].wait()
    out_h[nchunks - 1].wait()


@jax.jit
def _run(nums_t, table_flat):
    mesh = plsc.VectorSubcoreMesh(core_axis_name="c", subcore_axis_name="s")
    return pl.kernel(
        _body,
        mesh=mesh,
        compiler_params=pltpu.CompilerParams(needs_layout_passes=False),
        out_type=jax.ShapeDtypeStruct((_N_UNIT, _B), jnp.float32),
        scratch_types=[
            pltpu.VMEM((_SPLIT,), jnp.float32),
            pltpu.VMEM((_TABLE - _SPLIT,), jnp.float32),
            pltpu.VMEM((_R2, _CH), jnp.int32),
            pltpu.VMEM((_R2, _CH), jnp.int32),
            pltpu.VMEM((_R2, _CH), jnp.float32),
            pltpu.VMEM((_R2, _CH), jnp.float32),
            pltpu.VMEM_SHARED((_TABLE,), jnp.float32),
            pltpu.SemaphoreType.DMA,
            pltpu.SemaphoreType.DMA,
            pltpu.SemaphoreType.DMA,
            pltpu.SemaphoreType.DMA,
            pltpu.SemaphoreType.DMA,
        ],
    )(nums_t, table_flat)


def kernel(unit_nums, embed_weight):
    out_t = _run(unit_nums.T, embed_weight.reshape(-1))
    return out_t.T


# speedup vs baseline: 502.9662x; 1.0726x over previous
import functools

import jax
import jax.numpy as jnp
from jax import lax
from jax.experimental import pallas as pl
from jax.experimental.pallas import tpu as pltpu
from jax.experimental.pallas import tpu_sc as plsc

_N_UNIT = 100
_MAX_UNIT_SIZE = 800
_TABLE = _N_UNIT * _MAX_UNIT_SIZE
_B = 16384
_NC = 2
_NS = 16
_NW = _NC * _NS
_COLS_W = _B // _NW
_L = 16
_CH = 128
_NCH = _COLS_W // _CH
_R1 = 48
_R2 = _N_UNIT - _R1
_RB = 56
_SPLIT = _R1 * _MAX_UNIT_SIZE


def _body(nums_hbm, table_hbm, out_hbm, tv1, tv2, xb, ob, shared,
          sin0, sin1, sout0, sout1, semt):
    cid = lax.axis_index("c")
    sid = lax.axis_index("s")
    wid = sid * _NC + cid
    base = wid * _COLS_W

    sins = (sin0, sin1)
    souts = (sout0, sout1)
    rows = (_R1, _R2)

    def in_slice(phase, cc, nr):
        return nums_hbm.at[
            pl.ds(phase * _R1, nr), pl.ds(base + cc * _CH, _CH)
        ]

    def out_slice(phase, cc, nr):
        return out_hbm.at[
            pl.ds(phase * _R1, nr), pl.ds(base + cc * _CH, _CH)
        ]

    pltpu.async_copy(in_slice(0, 0, _R1), xb.at[0, pl.ds(0, _R1), :], sin0)
    pltpu.async_copy(in_slice(0, 1, _R1), xb.at[1, pl.ds(0, _R1), :], sin1)

    @pl.when(sid == 0)
    def _():
        pltpu.sync_copy(table_hbm, shared)

    plsc.subcore_barrier()
    pltpu.sync_copy(shared.at[pl.ds(0, _SPLIT)], tv1)
    h2 = pltpu.async_copy(shared.at[pl.ds(_SPLIT, _TABLE - _SPLIT)], tv2, semt)

    iota = lax.broadcasted_iota(jnp.int32, (_L,), 0)
    zeros = jnp.zeros((_L,), jnp.float32)

    def zero_mask_slots(tv, nr):
        for g in range((nr + _L - 1) // _L):
            unit = iota + g * _L
            plsc.store_scatter(
                tv, [unit * _MAX_UNIT_SIZE], zeros, mask=unit < nr
            )

    zero_mask_slots(tv1, _R1)

    def run_phase(phase, tv, nr):
        def chunk(cc, carry):
            s = lax.rem(cc, 2)
            xs = xb.at[s, pl.ds(0, nr), :]
            os_ = ob.at[s, pl.ds(0, nr), :]

            @pl.when(s == 0)
            def _():
                pltpu.make_async_copy(in_slice(phase, cc, nr), xs, sin0).wait()

            @pl.when(s == 1)
            def _():
                pltpu.make_async_copy(in_slice(phase, cc, nr), xs, sin1).wait()

            @pl.when(cc >= 2)
            def _():
                @pl.when(s == 0)
                def _():
                    pltpu.make_async_copy(
                        os_, out_slice(phase, cc - 2, nr), sout0
                    ).wait()

                @pl.when(s == 1)
                def _():
                    pltpu.make_async_copy(
                        os_, out_slice(phase, cc - 2, nr), sout1
                    ).wait()

            @plsc.parallel_loop(0, nr, 1, unroll=2)
            def _(r):
                shift = r * _MAX_UNIT_SIZE
                for k in range(_CH // _L):
                    off = pl.ds(k * _L, _L)
                    idx = xb[s, r, off] + shift
                    v = plsc.load_gather(tv, [idx])
                    ob[s, r, off] = jnp.maximum(v, 0.0)

            @pl.when(s == 0)
            def _():
                pltpu.async_copy(os_, out_slice(phase, cc, nr), sout0)

            @pl.when(s == 1)
            def _():
                pltpu.async_copy(os_, out_slice(phase, cc, nr), sout1)

            @pl.when(cc + 2 < _NCH)
            def _():
                @pl.when(s == 0)
                def _():
                    pltpu.async_copy(in_slice(phase, cc + 2, nr), xs, sin0)

                @pl.when(s == 1)
                def _():
                    pltpu.async_copy(in_slice(phase, cc + 2, nr), xs, sin1)

            return carry

        lax.fori_loop(0, _NCH, chunk, 0)
        for cc in (_NCH - 2, _NCH - 1):
            s = cc % 2
            pltpu.make_async_copy(
                ob.at[s, pl.ds(0, nr), :],
                out_slice(phase, cc, nr),
                souts[s],
            ).wait()

    run_phase(0, tv1, _R1)

    h2.wait()
    zero_mask_slots(tv2, _R2)
    pltpu.async_copy(in_slice(1, 0, _R2), xb.at[0, pl.ds(0, _R2), :], sin0)
    pltpu.async_copy(in_slice(1, 1, _R2), xb.at[1, pl.ds(0, _R2), :], sin1)
    run_phase(1, tv2, _R2)


@jax.jit
def _run(nums_t, table_flat):
    mesh = plsc.VectorSubcoreMesh(core_axis_name="c", subcore_axis_name="s")
    return pl.kernel(
        _body,
        mesh=mesh,
        compiler_params=pltpu.CompilerParams(needs_layout_passes=False),
        out_type=jax.ShapeDtypeStruct((_N_UNIT, _B), jnp.float32),
        scratch_types=[
            pltpu.VMEM((_SPLIT,), jnp.float32),
            pltpu.VMEM((_TABLE - _SPLIT,), jnp.float32),
            pltpu.VMEM((2, _RB, _CH), jnp.int32),
            pltpu.VMEM((2, _RB, _CH), jnp.float32),
            pltpu.VMEM_SHARED((_TABLE,), jnp.float32),
            pltpu.SemaphoreType.DMA,
            pltpu.SemaphoreType.DMA,
            pltpu.SemaphoreType.DMA,
            pltpu.SemaphoreType.DMA,
            pltpu.SemaphoreType.DMA,
        ],
    )(nums_t, table_flat)


def kernel(unit_nums, embed_weight):
    out_t = _run(unit_nums.T, embed_weight[:, 0])
    return out_t.T
